# baseline (speedup 1.0000x reference)
import jax
import jax.numpy as jnp
from jax.experimental import pallas as pl
from jax.experimental.pallas import tpu as pltpu

L_IN = 244
L1P = 122
L2P = 61
C1, C2, H, OUT = 16, 32, 128, 1
CS2 = 64
FLATPAD = C2 * CS2
B = 1024


def _cnn_kernel(x_ref, w1_ref, b1_ref, w2_ref, b2_ref,
                wf1_ref, bf1_ref, wf2_ref, bf2_ref,
                o_ref, p2_ref):
    EB = L2P * B
    zb1 = jnp.zeros((1, B), jnp.float32)

    xt = jnp.transpose(x_ref[...])
    xt4 = xt.reshape(L2P, 4, B)
    u0 = xt4[:, 0, :].reshape(1, EB)
    u1 = xt4[:, 1, :].reshape(1, EB)
    u2 = xt4[:, 2, :].reshape(1, EB)
    u3 = xt4[:, 3, :].reshape(1, EB)

    a1 = jnp.concatenate([
        jnp.concatenate([zb1, u3[:, 0:EB - B], u1], axis=1),
        jnp.concatenate([u0, u2], axis=1),
        jnp.concatenate([u1, u3], axis=1),
        jnp.concatenate([u2, u0[:, B:EB], zb1], axis=1),
    ], axis=0)

    h1e = jnp.dot(w1_ref[...], a1[0:3],
                  preferred_element_type=jnp.float32)
    h1o = jnp.dot(w1_ref[...], a1[1:4],
                  preferred_element_type=jnp.float32)
    b1c = b1_ref[...]
    p1 = jnp.maximum(jnp.maximum(h1e + b1c, 0.0),
                     jnp.maximum(h1o + b1c, 0.0))
    pev = p1[:, 0:EB]
    pod = p1[:, EB:2 * EB]
    zb = jnp.zeros((C1, B), jnp.float32)

    a2e = jnp.concatenate([
        jnp.concatenate([zb, pod[:, 0:EB - B]], axis=1),
        pev,
        pod,
    ], axis=0)
    a2o = jnp.concatenate([
        pev,
        pod,
        jnp.concatenate([pev[:, B:EB], zb], axis=1),
    ], axis=0)
    h2e = jnp.dot(w2_ref[...], a2e,
                  preferred_element_type=jnp.float32)
    h2o = jnp.dot(w2_ref[...], a2o,
                  preferred_element_type=jnp.float32)
    b2c = b2_ref[...]
    p2 = jnp.maximum(jnp.maximum(h2e + b2c, 0.0),
                     jnp.maximum(h2o + b2c, 0.0))

    p2_ref[:, 0:L2P, :] = p2.reshape(C2, L2P, B)
    p2_ref[:, L2P:CS2, :] = jnp.zeros((C2, CS2 - L2P, B), jnp.float32)

    flat = p2_ref[...].reshape(FLATPAD, B)
    h3 = jnp.dot(wf1_ref[...], flat,
                 preferred_element_type=jnp.float32)
    h3 = jnp.maximum(h3 + bf1_ref[...], 0.0)
    out = jnp.dot(wf2_ref[...], h3,
                  preferred_element_type=jnp.float32) + bf2_ref[...]
    o_ref[...] = out.reshape(1, 1, B)


def kernel(x, w1, b1, w2, b2, wf1, bf1, wf2, bf2):
    N = x.shape[0]
    NB = pl.cdiv(N, B)
    Npad = NB * B

    xs = x[:, 0, :].astype(jnp.float32)
    if Npad != N:
        xs = jnp.pad(xs, ((0, Npad - N), (0, 0)))

    w1k = w1[:, 0, :].astype(jnp.float32)
    b1k = b1.reshape(C1, 1).astype(jnp.float32)
    w2k = jnp.transpose(w2.astype(jnp.float32), (0, 2, 1)).reshape(C2, C1 * 3)
    b2k = b2.reshape(C2, 1).astype(jnp.float32)
    wf1k = jnp.pad(wf1.reshape(H, C2, L2P).astype(jnp.float32),
                   ((0, 0), (0, 0), (0, CS2 - L2P))).reshape(H, FLATPAD)
    bf1k = bf1.reshape(H, 1).astype(jnp.float32)
    wf2k = wf2.astype(jnp.float32)
    bf2k = bf2.reshape(1, 1).astype(jnp.float32)

    const = lambda n: (0, 0)

    out = pl.pallas_call(
        _cnn_kernel,
        out_shape=jax.ShapeDtypeStruct((NB, 1, B), jnp.float32),
        grid=(NB,),
        in_specs=[
            pl.BlockSpec((B, L_IN), lambda n: (n, 0)),
            pl.BlockSpec((C1, 3), const),
            pl.BlockSpec((C1, 1), const),
            pl.BlockSpec((C2, C1 * 3), const),
            pl.BlockSpec((C2, 1), const),
            pl.BlockSpec((H, FLATPAD), const),
            pl.BlockSpec((H, 1), const),
            pl.BlockSpec((1, H), const),
            pl.BlockSpec((1, 1), const),
        ],
        out_specs=pl.BlockSpec((1, 1, B), lambda n: (n, 0, 0)),
        scratch_shapes=[
            pltpu.VMEM((C2, CS2, B), jnp.float32),
        ],
        compiler_params=pltpu.CompilerParams(
            dimension_semantics=("parallel",)),
    )(xs, w1k, b1k, w2k, b2k, wf1k, bf1k, wf2k, bf2k)

    return out.reshape(Npad)[:N].reshape(N, 1)


# speedup vs baseline: 1.0274x; 1.0274x over previous
import jax
import jax.numpy as jnp
from jax.experimental import pallas as pl
from jax.experimental.pallas import tpu as pltpu

L_IN = 244
L1P = 122
L2P = 61
C1, C2, H, OUT = 16, 32, 128, 1
CS2 = 64
FLATPAD = C2 * CS2
B = 512


def _cnn_kernel(x_ref, w1_ref, b1_ref, w2_ref, b2_ref,
                wf1_ref, bf1_ref, wf2_ref, bf2_ref,
                o_ref, p2_ref):
    EB = L2P * B
    zb1 = jnp.zeros((1, B), jnp.float32)

    xt = jnp.transpose(x_ref[...])
    xt4 = xt.reshape(L2P, 4, B)
    u0 = xt4[:, 0, :].reshape(1, EB)
    u1 = xt4[:, 1, :].reshape(1, EB)
    u2 = xt4[:, 2, :].reshape(1, EB)
    u3 = xt4[:, 3, :].reshape(1, EB)

    a1 = jnp.concatenate([
        jnp.concatenate([zb1, u3[:, 0:EB - B], u1], axis=1),
        jnp.concatenate([u0, u2], axis=1),
        jnp.concatenate([u1, u3], axis=1),
        jnp.concatenate([u2, u0[:, B:EB], zb1], axis=1),
    ], axis=0)

    h1e = jnp.dot(w1_ref[...], a1[0:3],
                  preferred_element_type=jnp.float32)
    h1o = jnp.dot(w1_ref[...], a1[1:4],
                  preferred_element_type=jnp.float32)
    b1c = b1_ref[...]
    p1 = jnp.maximum(jnp.maximum(h1e + b1c, 0.0),
                     jnp.maximum(h1o + b1c, 0.0))
    pev = p1[:, 0:EB]
    pod = p1[:, EB:2 * EB]
    zb = jnp.zeros((C1, B), jnp.float32)

    a2e = jnp.concatenate([
        jnp.concatenate([zb, pod[:, 0:EB - B]], axis=1),
        pev,
        pod,
    ], axis=0)
    a2o = jnp.concatenate([
        pev,
        pod,
        jnp.concatenate([pev[:, B:EB], zb], axis=1),
    ], axis=0)
    h2e = jnp.dot(w2_ref[...], a2e,
                  preferred_element_type=jnp.float32)
    h2o = jnp.dot(w2_ref[...], a2o,
                  preferred_element_type=jnp.float32)
    b2c = b2_ref[...]
    p2 = jnp.maximum(jnp.maximum(h2e + b2c, 0.0),
                     jnp.maximum(h2o + b2c, 0.0))

    p2_ref[:, 0:L2P, :] = p2.reshape(C2, L2P, B)
    p2_ref[:, L2P:CS2, :] = jnp.zeros((C2, CS2 - L2P, B), jnp.float32)

    flat = p2_ref[...].reshape(FLATPAD, B)
    h3 = jnp.dot(wf1_ref[...], flat,
                 preferred_element_type=jnp.float32)
    h3 = jnp.maximum(h3 + bf1_ref[...], 0.0)
    out = jnp.dot(wf2_ref[...], h3,
                  preferred_element_type=jnp.float32) + bf2_ref[...]
    o_ref[...] = out.reshape(1, 1, B)


def kernel(x, w1, b1, w2, b2, wf1, bf1, wf2, bf2):
    N = x.shape[0]
    NB = pl.cdiv(N, B)
    Npad = NB * B

    xs = x[:, 0, :].astype(jnp.float32)
    if Npad != N:
        xs = jnp.pad(xs, ((0, Npad - N), (0, 0)))

    w1k = w1[:, 0, :].astype(jnp.float32)
    b1k = b1.reshape(C1, 1).astype(jnp.float32)
    w2k = jnp.transpose(w2.astype(jnp.float32), (0, 2, 1)).reshape(C2, C1 * 3)
    b2k = b2.reshape(C2, 1).astype(jnp.float32)
    wf1k = jnp.pad(wf1.reshape(H, C2, L2P).astype(jnp.float32),
                   ((0, 0), (0, 0), (0, CS2 - L2P))).reshape(H, FLATPAD)
    bf1k = bf1.reshape(H, 1).astype(jnp.float32)
    wf2k = wf2.astype(jnp.float32)
    bf2k = bf2.reshape(1, 1).astype(jnp.float32)

    const = lambda n: (0, 0)

    out = pl.pallas_call(
        _cnn_kernel,
        out_shape=jax.ShapeDtypeStruct((NB, 1, B), jnp.float32),
        grid=(NB,),
        in_specs=[
            pl.BlockSpec((B, L_IN), lambda n: (n, 0)),
            pl.BlockSpec((C1, 3), const),
            pl.BlockSpec((C1, 1), const),
            pl.BlockSpec((C2, C1 * 3), const),
            pl.BlockSpec((C2, 1), const),
            pl.BlockSpec((H, FLATPAD), const),
            pl.BlockSpec((H, 1), const),
            pl.BlockSpec((1, H), const),
            pl.BlockSpec((1, 1), const),
        ],
        out_specs=pl.BlockSpec((1, 1, B), lambda n: (n, 0, 0)),
        scratch_shapes=[
            pltpu.VMEM((C2, CS2, B), jnp.float32),
        ],
        compiler_params=pltpu.CompilerParams(
            dimension_semantics=("parallel",)),
    )(xs, w1k, b1k, w2k, b2k, wf1k, bf1k, wf2k, bf2k)

    return out.reshape(Npad)[:N].reshape(N, 1)
